# Initial kernel scaffold; baseline (speedup 1.0000x reference)
import jax
import jax.numpy as jnp
from jax.experimental import pallas as pl


def kernel(x, edge_index, edge_type, batch, emb, W1, root1, b1, W2, root2, b2, linW, linb):
    raise NotImplementedError("write your pallas kernel here")


# speedup vs baseline: 1.8720x; 1.8720x over previous
import functools

import jax
import jax.numpy as jnp
from jax.experimental import pallas as pl
from jax.experimental.pallas import tpu as pltpu

_NREL = 3
_NG = 64


def _rgcn_dense_body(h_ref, s_ref, cnt_ref, W_ref, root_ref, b_ref, out_ref):
    h = h_ref[...]
    acc = jnp.dot(h, root_ref[...], preferred_element_type=jnp.float32)
    acc = acc + b_ref[...]
    s = s_ref[...]
    rec = 1.0 / jnp.maximum(cnt_ref[...], 1.0)
    for r in range(_NREL):
        agg = jnp.dot(s[r], W_ref[r], preferred_element_type=jnp.float32)
        acc = acc + agg * rec[:, r : r + 1]
    out_ref[...] = jnp.maximum(acc, 0.0)


def _rgcn_pool_body(n_valid, blk,
                    h_ref, s_ref, cnt_ref, W_ref, root_ref, b_ref,
                    batch_ref, linW_ref, linb_ref, out_ref, g_scr):
    i = pl.program_id(0)
    nb = pl.num_programs(0)

    h = h_ref[...]
    acc = jnp.dot(h, root_ref[...], preferred_element_type=jnp.float32)
    acc = acc + b_ref[...]
    s = s_ref[...]
    rec = 1.0 / jnp.maximum(cnt_ref[...], 1.0)
    for r in range(_NREL):
        agg = jnp.dot(s[r], W_ref[r], preferred_element_type=jnp.float32)
        acc = acc + agg * rec[:, r : r + 1]
    h2 = jnp.maximum(acc, 0.0)

    @pl.when(i == 0)
    def _init():
        g_scr[...] = jnp.full(g_scr.shape, -jnp.inf, jnp.float32)

    rows = i * blk + jax.lax.broadcasted_iota(jnp.int32, (blk, 1), 0)
    valid = rows < n_valid
    h2m = jnp.where(valid, h2, -jnp.inf)
    b_ids = batch_ref[...]
    for g in range(_NG):
        red = jnp.max(
            jnp.where(b_ids == g, h2m, -jnp.inf), axis=0, keepdims=True
        )
        g_scr[g : g + 1, :] = jnp.maximum(g_scr[g : g + 1, :], red)

    @pl.when(i == nb - 1)
    def _final():
        g = g_scr[...]
        out_ref[...] = (
            jnp.dot(g, linW_ref[...], preferred_element_type=jnp.float32)
            + linb_ref[...]
        )


def kernel(x, edge_index, edge_type, batch, emb, W1, root1, b1,
           W2, root2, b2, linW, linb):
    N = x.shape[0]
    E = edge_type.shape[0]
    EMB = emb.shape[1]
    HID = root1.shape[1]
    NCLASS = linW.shape[1]

    src = edge_index[0].astype(jnp.int32)
    dst = edge_index[1].astype(jnp.int32)
    et = edge_type.astype(jnp.int32)
    seg = dst * _NREL + et

    h0 = jnp.take(emb, x.astype(jnp.int32), axis=0)
    cnt = jax.ops.segment_sum(
        jnp.ones((E,), jnp.float32), seg, num_segments=_NREL * N
    ).reshape(N, _NREL)
    s1 = jax.ops.segment_sum(
        jnp.take(h0, src, axis=0), seg, num_segments=_NREL * N
    ).reshape(N, _NREL, EMB).transpose(1, 0, 2)

    blk1 = 512
    blk2 = 256
    Np = ((N + blk1 - 1) // blk1) * blk1
    pad = Np - N
    h0p = jnp.pad(h0, ((0, pad), (0, 0)))
    s1p = jnp.pad(s1, ((0, 0), (0, pad), (0, 0)))
    cntp = jnp.pad(cnt, ((0, pad), (0, 0)))
    b1r = b1.reshape(1, HID)
    b2r = b2.reshape(1, HID)

    h1 = pl.pallas_call(
        _rgcn_dense_body,
        grid=(Np // blk1,),
        in_specs=[
            pl.BlockSpec((blk1, EMB), lambda i: (i, 0)),
            pl.BlockSpec((_NREL, blk1, EMB), lambda i: (0, i, 0)),
            pl.BlockSpec((blk1, _NREL), lambda i: (i, 0)),
            pl.BlockSpec((_NREL, EMB, HID), lambda i: (0, 0, 0)),
            pl.BlockSpec((EMB, HID), lambda i: (0, 0)),
            pl.BlockSpec((1, HID), lambda i: (0, 0)),
        ],
        out_specs=pl.BlockSpec((blk1, HID), lambda i: (i, 0)),
        out_shape=jax.ShapeDtypeStruct((Np, HID), jnp.float32),
    )(h0p, s1p, cntp, W1, root1, b1r)

    s2 = jax.ops.segment_sum(
        jnp.take(h1, src, axis=0), seg, num_segments=_NREL * N
    ).reshape(N, _NREL, HID).transpose(1, 0, 2)
    s2p = jnp.pad(s2, ((0, 0), (0, pad), (0, 0)))

    batchp = jnp.pad(batch.astype(jnp.int32), (0, pad)).reshape(Np, 1)

    out = pl.pallas_call(
        functools.partial(_rgcn_pool_body, N, blk2),
        grid=(Np // blk2,),
        in_specs=[
            pl.BlockSpec((blk2, HID), lambda i: (i, 0)),
            pl.BlockSpec((_NREL, blk2, HID), lambda i: (0, i, 0)),
            pl.BlockSpec((blk2, _NREL), lambda i: (i, 0)),
            pl.BlockSpec((_NREL, HID, HID), lambda i: (0, 0, 0)),
            pl.BlockSpec((HID, HID), lambda i: (0, 0)),
            pl.BlockSpec((1, HID), lambda i: (0, 0)),
            pl.BlockSpec((blk2, 1), lambda i: (i, 0)),
            pl.BlockSpec((HID, NCLASS), lambda i: (0, 0)),
            pl.BlockSpec((1, NCLASS), lambda i: (0, 0)),
        ],
        out_specs=pl.BlockSpec((_NG, NCLASS), lambda i: (0, 0)),
        out_shape=jax.ShapeDtypeStruct((_NG, NCLASS), jnp.float32),
        scratch_shapes=[pltpu.VMEM((_NG, HID), jnp.float32)],
    )(h1, s2p, cntp, W2, root2, b2r, batchp, linW, linb.reshape(1, NCLASS))

    return out


# speedup vs baseline: 1.8859x; 1.0074x over previous
import functools

import jax
import jax.numpy as jnp
from jax.experimental import pallas as pl
from jax.experimental.pallas import tpu as pltpu

_NREL = 3
_NG = 64


def _rgcn_dense_body(h_ref, s_ref, cnt_ref, W_ref, root_ref, b_ref, out_ref):
    h = h_ref[...]
    acc = jnp.dot(h, root_ref[...], preferred_element_type=jnp.float32)
    acc = acc + b_ref[...]
    s = s_ref[...]
    rec = 1.0 / jnp.maximum(cnt_ref[...], 1.0)
    for r in range(_NREL):
        agg = jnp.dot(s[r], W_ref[r], preferred_element_type=jnp.float32)
        acc = acc + agg * rec[:, r : r + 1]
    out_ref[...] = jnp.maximum(acc, 0.0)


def _rgcn_pool_body(n_valid, blk,
                    h_ref, s_ref, cnt_ref, W_ref, root_ref, b_ref,
                    batch_ref, linW_ref, linb_ref, out_ref, g_scr):
    i = pl.program_id(0)
    nb = pl.num_programs(0)

    h = h_ref[...]
    acc = jnp.dot(h, root_ref[...], preferred_element_type=jnp.float32)
    acc = acc + b_ref[...]
    s = s_ref[...]
    rec = 1.0 / jnp.maximum(cnt_ref[...], 1.0)
    for r in range(_NREL):
        agg = jnp.dot(s[r], W_ref[r], preferred_element_type=jnp.float32)
        acc = acc + agg * rec[:, r : r + 1]
    h2 = jnp.maximum(acc, 0.0)

    @pl.when(i == 0)
    def _init():
        g_scr[...] = jnp.full(g_scr.shape, -jnp.inf, jnp.float32)

    rows = i * blk + jax.lax.broadcasted_iota(jnp.int32, (blk, 1), 0)
    valid = rows < n_valid
    h2m = jnp.where(valid, h2, -jnp.inf)
    b_ids = batch_ref[...]
    for g in range(_NG):
        red = jnp.max(
            jnp.where(b_ids == g, h2m, -jnp.inf), axis=0, keepdims=True
        )
        g_scr[g : g + 1, :] = jnp.maximum(g_scr[g : g + 1, :], red)

    @pl.when(i == nb - 1)
    def _final():
        g = g_scr[...]
        out_ref[...] = (
            jnp.dot(g, linW_ref[...], preferred_element_type=jnp.float32)
            + linb_ref[...]
        )


def kernel(x, edge_index, edge_type, batch, emb, W1, root1, b1,
           W2, root2, b2, linW, linb):
    N = x.shape[0]
    E = edge_type.shape[0]
    EMB = emb.shape[1]
    HID = root1.shape[1]
    NCLASS = linW.shape[1]

    src = edge_index[0].astype(jnp.int32)
    dst = edge_index[1].astype(jnp.int32)
    et = edge_type.astype(jnp.int32)
    seg = et * N + dst

    h0 = jnp.take(emb, x.astype(jnp.int32), axis=0)
    h0aug = jnp.concatenate([h0, jnp.ones((N, 1), jnp.float32)], axis=1)
    sc1 = jax.ops.segment_sum(
        jnp.take(h0aug, src, axis=0), seg, num_segments=_NREL * N
    ).reshape(_NREL, N, EMB + 1)
    s1 = sc1[:, :, :EMB]
    cnt = sc1[:, :, EMB].T

    blk1 = 512
    blk2 = 256
    Np = ((N + blk1 - 1) // blk1) * blk1
    pad = Np - N
    h0p = jnp.pad(h0, ((0, pad), (0, 0)))
    s1p = jnp.pad(s1, ((0, 0), (0, pad), (0, 0)))
    cntp = jnp.pad(cnt, ((0, pad), (0, 0)))
    b1r = b1.reshape(1, HID)
    b2r = b2.reshape(1, HID)

    h1 = pl.pallas_call(
        _rgcn_dense_body,
        grid=(Np // blk1,),
        in_specs=[
            pl.BlockSpec((blk1, EMB), lambda i: (i, 0)),
            pl.BlockSpec((_NREL, blk1, EMB), lambda i: (0, i, 0)),
            pl.BlockSpec((blk1, _NREL), lambda i: (i, 0)),
            pl.BlockSpec((_NREL, EMB, HID), lambda i: (0, 0, 0)),
            pl.BlockSpec((EMB, HID), lambda i: (0, 0)),
            pl.BlockSpec((1, HID), lambda i: (0, 0)),
        ],
        out_specs=pl.BlockSpec((blk1, HID), lambda i: (i, 0)),
        out_shape=jax.ShapeDtypeStruct((Np, HID), jnp.float32),
    )(h0p, s1p, cntp, W1, root1, b1r)

    s2 = jax.ops.segment_sum(
        jnp.take(h1, src, axis=0), seg, num_segments=_NREL * N
    ).reshape(_NREL, N, HID)
    s2p = jnp.pad(s2, ((0, 0), (0, pad), (0, 0)))

    batchp = jnp.pad(batch.astype(jnp.int32), (0, pad)).reshape(Np, 1)

    out = pl.pallas_call(
        functools.partial(_rgcn_pool_body, N, blk2),
        grid=(Np // blk2,),
        in_specs=[
            pl.BlockSpec((blk2, HID), lambda i: (i, 0)),
            pl.BlockSpec((_NREL, blk2, HID), lambda i: (0, i, 0)),
            pl.BlockSpec((blk2, _NREL), lambda i: (i, 0)),
            pl.BlockSpec((_NREL, HID, HID), lambda i: (0, 0, 0)),
            pl.BlockSpec((HID, HID), lambda i: (0, 0)),
            pl.BlockSpec((1, HID), lambda i: (0, 0)),
            pl.BlockSpec((blk2, 1), lambda i: (i, 0)),
            pl.BlockSpec((HID, NCLASS), lambda i: (0, 0)),
            pl.BlockSpec((1, NCLASS), lambda i: (0, 0)),
        ],
        out_specs=pl.BlockSpec((_NG, NCLASS), lambda i: (0, 0)),
        out_shape=jax.ShapeDtypeStruct((_NG, NCLASS), jnp.float32),
        scratch_shapes=[pltpu.VMEM((_NG, HID), jnp.float32)],
    )(h1, s2p, cntp, W2, root2, b2r, batchp, linW, linb.reshape(1, NCLASS))

    return out
